# Initial kernel scaffold; baseline (speedup 1.0000x reference)
import functools

import jax
import jax.numpy as jnp
from jax import lax
from jax.experimental import pallas as pl
from jax.experimental.pallas import tpu as pltpu
from jax.experimental.pallas import tpu_sc as plsc

N = 10000
E = 160000
F_IN = 256
H = 512
C = 64
L = 5
G = 64

CB = 128
NSUB = 16
NCORE = 2
CHUNK = 128
EPP = 10112
EPAD = EPP * NSUB
NCHUNK = EPP // CHUNK
ROWS_PER_SUB = N // NSUB
NACC = N + NSUB

NB = 1000


@functools.cache
def _make_agg(num_blocks):
  mesh = plsc.VectorSubcoreMesh(core_axis_name="c", subcore_axis_name="s")
  out_type = tuple(
      jax.ShapeDtypeStruct((N, CB), jnp.float32) for _ in range(num_blocks))
  scratch = [
      pltpu.VMEM_SHARED((NACC, CB), jnp.float32),
      pltpu.VMEM((CHUNK,), jnp.int32),
      pltpu.VMEM((CHUNK,), jnp.int32),
      pltpu.VMEM((CHUNK, CB), jnp.float32),
      pltpu.SemaphoreType.DMA,
  ]

  def body(src_hbm, dst_hbm, *rest):
    h_refs = rest[:num_blocks]
    z_refs = rest[num_blocks:2 * num_blocks]
    acc, sidx, didx, gbuf, sem = rest[2 * num_blocks:]
    cid = lax.axis_index("c")
    sid = lax.axis_index("s")
    r0 = sid * ROWS_PER_SUB
    for b in range(num_blocks):
      @pl.when(cid == (b % NCORE))
      def _(b=b):
        hb = h_refs[b]
        zb = z_refs[b]
        pltpu.sync_copy(hb.at[pl.ds(r0, ROWS_PER_SUB)],
                        acc.at[pl.ds(r0, ROWS_PER_SUB)])
        plsc.subcore_barrier()

        def chunk(k, carry):
          base = sid * EPP + k * CHUNK
          pltpu.sync_copy(src_hbm.at[pl.ds(base, CHUNK)], sidx)
          pltpu.sync_copy(dst_hbm.at[pl.ds(base, CHUNK)], didx)
          pltpu.async_copy(hb.at[sidx], gbuf, sem).wait()
          pltpu.sync_copy(gbuf, acc.at[didx], add=True)
          return carry

        lax.fori_loop(0, NCHUNK, chunk, 0)
        plsc.subcore_barrier()
        pltpu.sync_copy(acc.at[pl.ds(r0, ROWS_PER_SUB)],
                        zb.at[pl.ds(r0, ROWS_PER_SUB)])
    return

  return pl.kernel(body, out_type, mesh=mesh, scratch_types=scratch)


@functools.cache
def _make_mlp(fin):
  nin = fin // CB
  grid = (N // NB,)
  in_specs = (
      [pl.BlockSpec((NB, CB), lambda i: (i, 0)) for _ in range(nin)] + [
          pl.BlockSpec((fin, H), lambda i: (0, 0)),
          pl.BlockSpec((1, H), lambda i: (0, 0)),
          pl.BlockSpec((H, H), lambda i: (0, 0)),
          pl.BlockSpec((1, H), lambda i: (0, 0)),
          pl.BlockSpec((NB, 1), lambda i: (i, 0)),
      ])
  out_specs = (
      [pl.BlockSpec((NB, CB), lambda i: (i, 0)) for _ in range(H // CB)] +
      [pl.BlockSpec((G, H), lambda i: (0, 0))])
  out_shape = (
      [jax.ShapeDtypeStruct((N, CB), jnp.float32) for _ in range(H // CB)] +
      [jax.ShapeDtypeStruct((G, H), jnp.float32)])

  def body(*refs):
    zrefs = refs[:nin]
    w1r, b1r, w2r, b2r, br = refs[nin:nin + 5]
    hrefs = refs[nin + 5:nin + 5 + H // CB]
    pr = refs[nin + 5 + H // CB]
    z = jnp.concatenate([r[...] for r in zrefs], axis=1)
    a = jnp.dot(z, w1r[...], preferred_element_type=jnp.float32) + b1r[...]
    a = jnp.where(a >= 0, a, 0.01 * a)
    h = jnp.dot(a, w2r[...], preferred_element_type=jnp.float32) + b2r[...]
    h = jnp.where(h >= 0, h, 0.01 * h)
    for j, hr in enumerate(hrefs):
      hr[...] = h[:, j * CB:(j + 1) * CB]
    oh = (br[...] == lax.broadcasted_iota(jnp.int32, (NB, G), 1)
          ).astype(jnp.float32)
    p = lax.dot_general(oh, h, (((0,), (0,)), ((), ())),
                        preferred_element_type=jnp.float32)
    i = pl.program_id(0)

    @pl.when(i == 0)
    def _():
      pr[...] = p

    @pl.when(i != 0)
    def _():
      pr[...] = pr[...] + p

  return pl.pallas_call(body, grid=grid, in_specs=in_specs,
                        out_specs=out_specs, out_shape=out_shape)


def _head(pooled, fc1w, fc1b, fc2w, fc2b):
  in_specs = (
      [pl.BlockSpec((G, H), lambda: (0, 0)) for _ in range(L)] + [
          pl.BlockSpec((H * L, H), lambda: (0, 0)),
          pl.BlockSpec((1, H), lambda: (0, 0)),
          pl.BlockSpec((H, C), lambda: (0, 0)),
          pl.BlockSpec((1, C), lambda: (0, 0)),
      ])

  def body(*refs):
    prefs = refs[:L]
    w1r, b1r, w2r, b2r, outr = refs[L:]
    g = b1r[...]
    for l, prr in enumerate(prefs):
      g = g + jnp.dot(prr[...], w1r[l * H:(l + 1) * H, :],
                      preferred_element_type=jnp.float32)
    g = jnp.where(g >= 0, g, 0.01 * g)
    outr[...] = jnp.dot(g, w2r[...],
                        preferred_element_type=jnp.float32) + b2r[...]

  return pl.pallas_call(
      body, grid=(), in_specs=in_specs,
      out_specs=pl.BlockSpec((G, C), lambda: (0, 0)),
      out_shape=jax.ShapeDtypeStruct((G, C), jnp.float32),
  )(*pooled, fc1w, fc1b, fc2w, fc2b)


def _fold_bn(w, b, g, bb, rm, rv):
  s = g / jnp.sqrt(rv + 1e-5)
  return w * s[None, :], ((b - rm) * s + bb)[None, :]


def kernel(x, edge_index, batch, params):
  src = edge_index[0]
  dst = edge_index[1]
  pad = EPAD - E
  src_p = jnp.concatenate([src, jnp.zeros((pad,), jnp.int32)])
  dst_p = jnp.concatenate([dst, jnp.full((pad,), N, jnp.int32)])
  batch2d = batch.reshape(N, 1)

  h_blocks = [x[:, j * CB:(j + 1) * CB] for j in range(F_IN // CB)]
  pooled = []
  for i in range(L):
    p = params['conv%d' % i]
    fin = F_IN if i == 0 else H
    w1, b1 = _fold_bn(p['W1'], p['b1'], p['bn1_g'], p['bn1_b'],
                      p['bn1_rm'], p['bn1_rv'])
    w2, b2 = _fold_bn(p['W2'], p['b2'], p['obn_g'], p['obn_b'],
                      p['obn_rm'], p['obn_rv'])
    z_blocks = _make_agg(fin // CB)(src_p, dst_p, *h_blocks)
    outs = _make_mlp(fin)(*z_blocks, w1, b1, w2, b2, batch2d)
    h_blocks = list(outs[:H // CB])
    pooled.append(outs[H // CB])

  return _head(pooled, params['fc1_W'], params['fc1_b'][None, :],
               params['fc2_W'], params['fc2_b'][None, :])


# speedup vs baseline: 2.7889x; 2.7889x over previous
import functools

import jax
import jax.numpy as jnp
from jax import lax
from jax.experimental import pallas as pl
from jax.experimental.pallas import tpu as pltpu
from jax.experimental.pallas import tpu_sc as plsc

N = 10000
E = 160000
F_IN = 256
H = 512
C = 64
L = 5
G = 64

CB = 128
NSUB = 16
NCORE = 2
CHUNK = 128
EPP = 10112
EPAD = EPP * NSUB
NCHUNK = EPP // CHUNK
ROWS_PER_SUB = 624
TAIL0 = NSUB * ROWS_PER_SUB
NTAIL = N - TAIL0
NACC = N + 16

NB = 1000


@functools.cache
def _make_agg(num_blocks):
  mesh = plsc.VectorSubcoreMesh(core_axis_name="c", subcore_axis_name="s")
  out_type = tuple(
      jax.ShapeDtypeStruct((N, CB), jnp.float32) for _ in range(num_blocks))
  scratch = [
      pltpu.VMEM_SHARED((NACC, CB), jnp.float32),
      pltpu.VMEM((CHUNK,), jnp.int32),
      pltpu.VMEM((CHUNK,), jnp.int32),
      pltpu.VMEM((CHUNK, CB), jnp.float32),
      pltpu.SemaphoreType.DMA,
  ]

  def body(src_hbm, dst_hbm, *rest):
    h_refs = rest[:num_blocks]
    z_refs = rest[num_blocks:2 * num_blocks]
    acc, sidx, didx, gbuf, sem = rest[2 * num_blocks:]
    cid = lax.axis_index("c")
    sid = lax.axis_index("s")
    r0 = pl.multiple_of(sid * ROWS_PER_SUB, 8)

    def copy_rows(a, b):
      pltpu.sync_copy(a.at[pl.ds(r0, ROWS_PER_SUB)],
                      b.at[pl.ds(r0, ROWS_PER_SUB)])
      @pl.when(sid == NSUB - 1)
      def _():
        pltpu.sync_copy(a.at[pl.ds(TAIL0, NTAIL)], b.at[pl.ds(TAIL0, NTAIL)])

    for b in range(num_blocks):
      @pl.when(cid == (b % NCORE))
      def _(b=b):
        hb = h_refs[b]
        zb = z_refs[b]
        copy_rows(hb, acc)
        plsc.subcore_barrier()

        def chunk(k, carry):
          base = pl.multiple_of(sid * EPP + k * CHUNK, 8)
          pltpu.sync_copy(src_hbm.at[pl.ds(base, CHUNK)], sidx)
          pltpu.sync_copy(dst_hbm.at[pl.ds(base, CHUNK)], didx)
          pltpu.async_copy(hb.at[sidx], gbuf, sem).wait()
          pltpu.sync_copy(gbuf, acc.at[didx], add=True)
          return carry

        lax.fori_loop(0, NCHUNK, chunk, 0)
        plsc.subcore_barrier()
        copy_rows(acc, zb)
    return

  return pl.kernel(body, out_type, mesh=mesh, scratch_types=scratch)


@functools.cache
def _make_mlp(fin):
  nin = fin // CB
  grid = (N // NB,)
  in_specs = (
      [pl.BlockSpec((NB, CB), lambda i: (i, 0)) for _ in range(nin)] + [
          pl.BlockSpec((fin, H), lambda i: (0, 0)),
          pl.BlockSpec((1, H), lambda i: (0, 0)),
          pl.BlockSpec((H, H), lambda i: (0, 0)),
          pl.BlockSpec((1, H), lambda i: (0, 0)),
          pl.BlockSpec((NB, 1), lambda i: (i, 0)),
      ])
  out_specs = (
      [pl.BlockSpec((NB, CB), lambda i: (i, 0)) for _ in range(H // CB)] +
      [pl.BlockSpec((G, H), lambda i: (0, 0))])
  out_shape = (
      [jax.ShapeDtypeStruct((N, CB), jnp.float32) for _ in range(H // CB)] +
      [jax.ShapeDtypeStruct((G, H), jnp.float32)])

  def body(*refs):
    zrefs = refs[:nin]
    w1r, b1r, w2r, b2r, br = refs[nin:nin + 5]
    hrefs = refs[nin + 5:nin + 5 + H // CB]
    pr = refs[nin + 5 + H // CB]
    z = jnp.concatenate([r[...] for r in zrefs], axis=1)
    a = jnp.dot(z, w1r[...], preferred_element_type=jnp.float32) + b1r[...]
    a = jnp.where(a >= 0, a, 0.01 * a)
    h = jnp.dot(a, w2r[...], preferred_element_type=jnp.float32) + b2r[...]
    h = jnp.where(h >= 0, h, 0.01 * h)
    for j, hr in enumerate(hrefs):
      hr[...] = h[:, j * CB:(j + 1) * CB]
    oh = (br[...] == lax.broadcasted_iota(jnp.int32, (NB, G), 1)
          ).astype(jnp.float32)
    p = lax.dot_general(oh, h, (((0,), (0,)), ((), ())),
                        preferred_element_type=jnp.float32)
    i = pl.program_id(0)

    @pl.when(i == 0)
    def _():
      pr[...] = p

    @pl.when(i != 0)
    def _():
      pr[...] = pr[...] + p

  return pl.pallas_call(body, grid=grid, in_specs=in_specs,
                        out_specs=out_specs, out_shape=out_shape)


def _head(pooled, fc1w, fc1b, fc2w, fc2b):
  in_specs = (
      [pl.BlockSpec((G, H), lambda: (0, 0)) for _ in range(L)] + [
          pl.BlockSpec((H * L, H), lambda: (0, 0)),
          pl.BlockSpec((1, H), lambda: (0, 0)),
          pl.BlockSpec((H, C), lambda: (0, 0)),
          pl.BlockSpec((1, C), lambda: (0, 0)),
      ])

  def body(*refs):
    prefs = refs[:L]
    w1r, b1r, w2r, b2r, outr = refs[L:]
    g = b1r[...]
    for l, prr in enumerate(prefs):
      g = g + jnp.dot(prr[...], w1r[l * H:(l + 1) * H, :],
                      preferred_element_type=jnp.float32)
    g = jnp.where(g >= 0, g, 0.01 * g)
    outr[...] = jnp.dot(g, w2r[...],
                        preferred_element_type=jnp.float32) + b2r[...]

  return pl.pallas_call(
      body, grid=(), in_specs=in_specs,
      out_specs=pl.BlockSpec((G, C), lambda: (0, 0)),
      out_shape=jax.ShapeDtypeStruct((G, C), jnp.float32),
  )(*pooled, fc1w, fc1b, fc2w, fc2b)


def _fold_bn(w, b, g, bb, rm, rv):
  s = g / jnp.sqrt(rv + 1e-5)
  return w * s[None, :], ((b - rm) * s + bb)[None, :]


def kernel(x, edge_index, batch, params):
  src = edge_index[0]
  dst = edge_index[1]
  pad = EPAD - E
  src_p = jnp.concatenate([src, jnp.zeros((pad,), jnp.int32)])
  dst_p = jnp.concatenate([dst, jnp.full((pad,), N, jnp.int32)])
  batch2d = batch.reshape(N, 1)

  h_blocks = [x[:, j * CB:(j + 1) * CB] for j in range(F_IN // CB)]
  pooled = []
  for i in range(L):
    p = params['conv%d' % i]
    fin = F_IN if i == 0 else H
    w1, b1 = _fold_bn(p['W1'], p['b1'], p['bn1_g'], p['bn1_b'],
                      p['bn1_rm'], p['bn1_rv'])
    w2, b2 = _fold_bn(p['W2'], p['b2'], p['obn_g'], p['obn_b'],
                      p['obn_rm'], p['obn_rv'])
    z_blocks = _make_agg(fin // CB)(src_p, dst_p, *h_blocks)
    outs = _make_mlp(fin)(*z_blocks, w1, b1, w2, b2, batch2d)
    h_blocks = list(outs[:H // CB])
    pooled.append(outs[H // CB])

  return _head(pooled, params['fc1_W'], params['fc1_b'][None, :],
               params['fc2_W'], params['fc2_b'][None, :])


# speedup vs baseline: 4.5653x; 1.6370x over previous
import functools

import jax
import jax.numpy as jnp
from jax import lax
from jax.experimental import pallas as pl
from jax.experimental.pallas import tpu as pltpu
from jax.experimental.pallas import tpu_sc as plsc

N = 10000
E = 160000
F_IN = 256
H = 512
C = 64
L = 5
G = 64

CB = 128
NSUB = 16
NCORE = 2
CHUNK = 128
EPP = 10112
EPAD = EPP * NSUB
NCHUNK = EPP // CHUNK
ROWS_PER_SUB = 624
TAIL0 = NSUB * ROWS_PER_SUB
NTAIL = N - TAIL0
NACC = N + 16

NB = 1000


@functools.cache
def _make_agg(num_blocks):
  mesh = plsc.VectorSubcoreMesh(core_axis_name="c", subcore_axis_name="s")
  out_type = tuple(
      jax.ShapeDtypeStruct((N, CB), jnp.float32) for _ in range(num_blocks))
  scratch = [
      pltpu.VMEM_SHARED((NACC, CB), jnp.float32),
      pltpu.VMEM((2, CHUNK), jnp.int32),
      pltpu.VMEM((NCHUNK, CHUNK), jnp.int32),
      pltpu.VMEM((CHUNK, CB), jnp.float32),
      pltpu.VMEM((CHUNK, CB), jnp.float32),
      pltpu.SemaphoreType.DMA,
      pltpu.SemaphoreType.DMA,
      pltpu.SemaphoreType.DMA,
      pltpu.SemaphoreType.DMA,
  ]

  def body(src_hbm, dst_hbm, *rest):
    h_refs = rest[:num_blocks]
    z_refs = rest[num_blocks:2 * num_blocks]
    acc, sring, didx, ga, gb, sema, semb, semsa, semsb = rest[2 * num_blocks:]
    cid = lax.axis_index("c")
    sid = lax.axis_index("s")
    r0 = pl.multiple_of(sid * ROWS_PER_SUB, 8)

    def copy_rows(a, b):
      pltpu.sync_copy(a.at[pl.ds(r0, ROWS_PER_SUB)],
                      b.at[pl.ds(r0, ROWS_PER_SUB)])
      @pl.when(sid == NSUB - 1)
      def _():
        pltpu.sync_copy(a.at[pl.ds(TAIL0, NTAIL)], b.at[pl.ds(TAIL0, NTAIL)])

    pltpu.sync_copy(dst_hbm.at[sid], didx)

    def idx_start(k, row, sem):
      pltpu.async_copy(src_hbm.at[sid, k], sring.at[row], sem)

    def idx_wait(row, sem):
      pltpu.make_async_copy(src_hbm.at[sid, 0], sring.at[row], sem).wait()

    for b in range(num_blocks):
      @pl.when(cid == (b % NCORE))
      def _(b=b):
        hb = h_refs[b]
        zb = z_refs[b]
        copy_rows(hb, acc)
        plsc.subcore_barrier()

        def start(row, buf, sem):
          pltpu.async_copy(hb.at[sring.at[row]], buf, sem)

        def wait(buf, sem):
          pltpu.make_async_copy(hb.at[sring.at[0]], buf, sem).wait()

        def scat(k, buf):
          pltpu.sync_copy(buf, acc.at[didx.at[k]], add=True)

        idx_start(0, 0, semsa)
        idx_wait(0, semsa)
        start(0, ga, sema)
        idx_start(1, 1, semsb)

        def pipe(i, carry):
          k0 = 2 * i
          idx_wait(1, semsb)
          start(1, gb, semb)
          wait(ga, sema)

          @pl.when(k0 + 2 < NCHUNK)
          def _():
            idx_start(k0 + 2, 0, semsa)
          scat(k0, ga)

          @pl.when(k0 + 2 < NCHUNK)
          def _():
            idx_wait(0, semsa)
            start(0, ga, sema)
          wait(gb, semb)

          @pl.when(k0 + 3 < NCHUNK)
          def _():
            idx_start(k0 + 3, 1, semsb)
          scat(k0 + 1, gb)
          return carry

        lax.fori_loop(0, NCHUNK // 2, pipe, 0)
        if NCHUNK % 2:
          wait(ga, sema)
          scat(NCHUNK - 1, ga)
        plsc.subcore_barrier()
        copy_rows(acc, zb)
    return

  return pl.kernel(body, out_type, mesh=mesh, scratch_types=scratch)


@functools.cache
def _make_mlp(fin):
  nin = fin // CB
  grid = (N // NB,)
  in_specs = (
      [pl.BlockSpec((NB, CB), lambda i: (i, 0)) for _ in range(nin)] + [
          pl.BlockSpec((fin, H), lambda i: (0, 0)),
          pl.BlockSpec((1, H), lambda i: (0, 0)),
          pl.BlockSpec((H, H), lambda i: (0, 0)),
          pl.BlockSpec((1, H), lambda i: (0, 0)),
          pl.BlockSpec((NB, 1), lambda i: (i, 0)),
      ])
  out_specs = (
      [pl.BlockSpec((NB, CB), lambda i: (i, 0)) for _ in range(H // CB)] +
      [pl.BlockSpec((G, H), lambda i: (0, 0))])
  out_shape = (
      [jax.ShapeDtypeStruct((N, CB), jnp.float32) for _ in range(H // CB)] +
      [jax.ShapeDtypeStruct((G, H), jnp.float32)])

  def body(*refs):
    zrefs = refs[:nin]
    w1r, b1r, w2r, b2r, br = refs[nin:nin + 5]
    hrefs = refs[nin + 5:nin + 5 + H // CB]
    pr = refs[nin + 5 + H // CB]
    z = jnp.concatenate([r[...] for r in zrefs], axis=1)
    a = jnp.dot(z, w1r[...], preferred_element_type=jnp.float32) + b1r[...]
    a = jnp.where(a >= 0, a, 0.01 * a)
    h = jnp.dot(a, w2r[...], preferred_element_type=jnp.float32) + b2r[...]
    h = jnp.where(h >= 0, h, 0.01 * h)
    for j, hr in enumerate(hrefs):
      hr[...] = h[:, j * CB:(j + 1) * CB]
    oh = (br[...] == lax.broadcasted_iota(jnp.int32, (NB, G), 1)
          ).astype(jnp.float32)
    p = lax.dot_general(oh, h, (((0,), (0,)), ((), ())),
                        preferred_element_type=jnp.float32)
    i = pl.program_id(0)

    @pl.when(i == 0)
    def _():
      pr[...] = p

    @pl.when(i != 0)
    def _():
      pr[...] = pr[...] + p

  return pl.pallas_call(body, grid=grid, in_specs=in_specs,
                        out_specs=out_specs, out_shape=out_shape)


def _head(pooled, fc1w, fc1b, fc2w, fc2b):
  in_specs = (
      [pl.BlockSpec((G, H), lambda: (0, 0)) for _ in range(L)] + [
          pl.BlockSpec((H * L, H), lambda: (0, 0)),
          pl.BlockSpec((1, H), lambda: (0, 0)),
          pl.BlockSpec((H, C), lambda: (0, 0)),
          pl.BlockSpec((1, C), lambda: (0, 0)),
      ])

  def body(*refs):
    prefs = refs[:L]
    w1r, b1r, w2r, b2r, outr = refs[L:]
    g = b1r[...]
    for l, prr in enumerate(prefs):
      g = g + jnp.dot(prr[...], w1r[l * H:(l + 1) * H, :],
                      preferred_element_type=jnp.float32)
    g = jnp.where(g >= 0, g, 0.01 * g)
    outr[...] = jnp.dot(g, w2r[...],
                        preferred_element_type=jnp.float32) + b2r[...]

  return pl.pallas_call(
      body, grid=(), in_specs=in_specs,
      out_specs=pl.BlockSpec((G, C), lambda: (0, 0)),
      out_shape=jax.ShapeDtypeStruct((G, C), jnp.float32),
  )(*pooled, fc1w, fc1b, fc2w, fc2b)


def _fold_bn(w, b, g, bb, rm, rv):
  s = g / jnp.sqrt(rv + 1e-5)
  return w * s[None, :], ((b - rm) * s + bb)[None, :]


def kernel(x, edge_index, batch, params):
  src = edge_index[0]
  dst = edge_index[1]
  pad = EPAD - E
  src_p = jnp.concatenate([src, jnp.zeros((pad,), jnp.int32)]
                          ).reshape(NSUB, NCHUNK, CHUNK)
  dst_p = jnp.concatenate([dst, jnp.full((pad,), N, jnp.int32)]
                          ).reshape(NSUB, NCHUNK, CHUNK)
  batch2d = batch.reshape(N, 1)

  h_blocks = [x[:, j * CB:(j + 1) * CB] for j in range(F_IN // CB)]
  pooled = []
  for i in range(L):
    p = params['conv%d' % i]
    fin = F_IN if i == 0 else H
    w1, b1 = _fold_bn(p['W1'], p['b1'], p['bn1_g'], p['bn1_b'],
                      p['bn1_rm'], p['bn1_rv'])
    w2, b2 = _fold_bn(p['W2'], p['b2'], p['obn_g'], p['obn_b'],
                      p['obn_rm'], p['obn_rv'])
    z_blocks = _make_agg(fin // CB)(src_p, dst_p, *h_blocks)
    outs = _make_mlp(fin)(*z_blocks, w1, b1, w2, b2, batch2d)
    h_blocks = list(outs[:H // CB])
    pooled.append(outs[H // CB])

  return _head(pooled, params['fc1_W'], params['fc1_b'][None, :],
               params['fc2_W'], params['fc2_b'][None, :])
